# baseline (speedup 1.0000x reference)
import functools

import jax
import jax.numpy as jnp
from jax import lax
from jax.experimental import pallas as pl
from jax.experimental.pallas import tpu as pltpu
from jax.experimental.pallas import tpu_sc as plsc

N = 10000
E = 320000
D = 128
G = 64

NC = 2
NS = 16
NW = NC * NS

CHUNK = 128
NCHUNKS = E // CHUNK
CPT = 80
E_PER_TILE = CPT * CHUNK
E_PAD = NW * E_PER_TILE
NPADE = E_PAD - E
PHASE = 40
FIRE = 8

ROW_STEP = 624
ROW_SPAN = 640


def _zero_vmem_2d(ref, nrows):
    zero = jnp.zeros((16,), jnp.float32)
    ncols = ref.shape[1]

    def body(i, c):
        for u in range(ncols // 16):
            ref[i, pl.ds(u * 16, 16)] = zero
        return c

    lax.fori_loop(0, nrows, body, 0)


def _fill_ones_vmem_2d(ref, nrows):
    one = jnp.ones((16,), jnp.float32)
    ncols = ref.shape[1]

    def body(i, c):
        for u in range(ncols // 16):
            ref[i, pl.ds(u * 16, 16)] = one
        return c

    lax.fori_loop(0, nrows, body, 0)


def _zero_spmem_slice(acc_sh, row0, nrows, zbuf, zrows):
    nfull = nrows // zrows
    rem = nrows - nfull * zrows
    for k in range(nfull):
        pltpu.sync_copy(zbuf, acc_sh.at[pl.ds(row0 + k * zrows, zrows)])
    if rem:
        pltpu.sync_copy(zbuf.at[pl.ds(0, rem)],
                        acc_sh.at[pl.ds(row0 + nfull * zrows, rem)])


def _stage_idx(idx_hbm, estart, idx2, sem):
    def round_(ko, c):
        for b in range(FIRE):
            j = ko * FIRE + b
            pltpu.async_copy(idx_hbm.at[pl.ds(estart + j * CHUNK, CHUNK)],
                             idx2.at[j], sem)
        for b in range(FIRE):
            j = ko * FIRE + b
            pltpu.make_async_copy(
                idx_hbm.at[pl.ds(estart + j * CHUNK, CHUNK)],
                idx2.at[j], sem).wait()
        return c

    lax.fori_loop(0, idx2.shape[0] // FIRE, round_, 0)


def _sc_deg_body(dst_hbm, out_hbm, ones_v, zbuf, didx2, acc_sh, semi, sems):
    cid = lax.axis_index("c")
    sid = lax.axis_index("s")
    wid = sid * NC + cid

    _zero_vmem_2d(zbuf, CHUNK)
    _zero_spmem_slice(acc_sh, sid * ROW_STEP, ROW_SPAN, zbuf, CHUNK)
    _fill_ones_vmem_2d(ones_v, CHUNK)

    estart = wid * E_PER_TILE
    _stage_idx(dst_hbm, estart, didx2, semi)
    plsc.subcore_barrier()

    def round_(ko, c):
        for b in range(FIRE):
            j = ko * FIRE + b
            pltpu.async_copy(ones_v, acc_sh.at[didx2.at[j]], sems, add=True)
        for b in range(FIRE):
            j = ko * FIRE + b
            pltpu.make_async_copy(ones_v, acc_sh.at[didx2.at[j]],
                                  sems).wait()
        return c

    lax.fori_loop(0, CPT // FIRE, round_, 0)

    plsc.subcore_barrier()
    row0 = sid * ROW_STEP
    pltpu.sync_copy(acc_sh.at[pl.ds(row0, ROW_SPAN)],
                    out_hbm.at[cid, pl.ds(row0, ROW_SPAN)])


def _sc_deg(dst):
    mesh = plsc.VectorSubcoreMesh(core_axis_name="c", subcore_axis_name="s")
    k = functools.partial(
        pl.kernel,
        out_type=jax.ShapeDtypeStruct((NC, N, 16), jnp.float32),
        mesh=mesh,
        scratch_types=[
            pltpu.VMEM((CHUNK, 16), jnp.float32),
            pltpu.VMEM((CHUNK, 16), jnp.float32),
            pltpu.VMEM((CPT, CHUNK), jnp.int32),
            pltpu.VMEM_SHARED((N, 16), jnp.float32),
            pltpu.SemaphoreType.DMA,
            pltpu.SemaphoreType.DMA,
        ],
    )(_sc_deg_body)
    return k(dst)


def _sc_agg_body(ht_hbm, src_hbm, dst_hbm, out_hbm,
                 sidxA, didxA, sidxB, didxB, rows0, rows1,
                 acc_sh, sem0, sem1, sems0, sems1):
    cid = lax.axis_index("c")
    sid = lax.axis_index("s")
    wid = cid * NS + sid

    _zero_vmem_2d(rows0, CHUNK)
    _zero_spmem_slice(acc_sh, sid * ROW_STEP, ROW_SPAN, rows0, CHUNK)
    plsc.subcore_barrier()

    estart = wid * E_PER_TILE

    def load_idx(j, sidx, didx):
        pltpu.sync_copy(src_hbm.at[pl.ds(estart + j * CHUNK, CHUNK)], sidx)
        pltpu.sync_copy(dst_hbm.at[pl.ds(estart + j * CHUNK, CHUNK)], didx)

    def start_g(sidx, buf, sem):
        pltpu.async_copy(ht_hbm.at[sidx], buf, sem)

    def wait_g(sidx, buf, sem):
        pltpu.make_async_copy(ht_hbm.at[sidx], buf, sem).wait()

    def start_s(didx, buf, sem):
        pltpu.async_copy(buf, acc_sh.at[didx], sem, add=True)

    def wait_s(didx, buf, sem):
        pltpu.make_async_copy(buf, acc_sh.at[didx], sem).wait()

    load_idx(0, sidxA, didxA)
    start_g(sidxA, rows0, sem0)
    load_idx(1, sidxB, didxB)
    start_g(sidxB, rows1, sem1)

    def body(k, c):
        j2 = 2 * k + 2
        wait_g(sidxA, rows0, sem0)
        start_s(didxA, rows0, sems0)
        wait_g(sidxB, rows1, sem1)
        start_s(didxB, rows1, sems1)
        wait_s(didxA, rows0, sems0)
        load_idx(j2, sidxA, didxA)
        start_g(sidxA, rows0, sem0)
        wait_s(didxB, rows1, sems1)
        load_idx(j2 + 1, sidxB, didxB)
        start_g(sidxB, rows1, sem1)
        return c

    lax.fori_loop(0, CPT // 2 - 1, body, 0)

    wait_g(sidxA, rows0, sem0)
    start_s(didxA, rows0, sems0)
    wait_g(sidxB, rows1, sem1)
    start_s(didxB, rows1, sems1)
    wait_s(didxA, rows0, sems0)
    wait_s(didxB, rows1, sems1)

    plsc.subcore_barrier()
    row0 = sid * ROW_STEP
    pltpu.sync_copy(acc_sh.at[pl.ds(row0, ROW_SPAN)],
                    out_hbm.at[cid, pl.ds(row0, ROW_SPAN)])


def _sc_agg(ht, src, dst):
    mesh = plsc.VectorSubcoreMesh(core_axis_name="c", subcore_axis_name="s")
    k = functools.partial(
        pl.kernel,
        out_type=jax.ShapeDtypeStruct((NC, N, D), jnp.float32),
        mesh=mesh,
        scratch_types=[
            pltpu.VMEM((CHUNK,), jnp.int32),
            pltpu.VMEM((CHUNK,), jnp.int32),
            pltpu.VMEM((CHUNK,), jnp.int32),
            pltpu.VMEM((CHUNK,), jnp.int32),
            pltpu.VMEM((CHUNK, D), jnp.float32),
            pltpu.VMEM((CHUNK, D), jnp.float32),
            pltpu.VMEM_SHARED((N, D), jnp.float32),
            pltpu.SemaphoreType.DMA,
            pltpu.SemaphoreType.DMA,
            pltpu.SemaphoreType.DMA,
            pltpu.SemaphoreType.DMA,
        ],
    )(_sc_agg_body)
    return k(ht, src, dst)


_HI = jax.lax.Precision.HIGHEST


def _tc1_body(x_ref, w_ref, degp_ref, dis_ref, ht_ref):
    degp = degp_ref[...]
    padc = (lax.broadcasted_iota(jnp.int32, (N, 1), 0) < NPADE)
    deg = (degp[0, :, 0:1] + degp[1, :, 0:1] + 1.0
           - padc.astype(jnp.float32))
    dis = lax.rsqrt(deg)
    h = jnp.dot(x_ref[...], w_ref[...],
                preferred_element_type=jnp.float32, precision=_HI)
    dis_ref[...] = dis
    ht_ref[...] = h * dis


def _tc1(x, W1, degp):
    return pl.pallas_call(
        _tc1_body,
        out_shape=[
            jax.ShapeDtypeStruct((N, 1), jnp.float32),
            jax.ShapeDtypeStruct((N, D), jnp.float32),
        ],
    )(x, W1, degp)


def _tc_mid_body(accp_ref, ht_ref, dis_ref, b_ref, w_ref, out_ref):
    ht = ht_ref[...]
    padm = (lax.broadcasted_iota(jnp.int32, (N, 1), 0) < NPADE)
    selfloop = jnp.where(padm, 0.0, ht)
    agg = accp_ref[0] + accp_ref[1] + selfloop
    z = jnp.maximum(dis_ref[...] * agg + b_ref[...], 0.0)
    h = jnp.dot(z, w_ref[...],
                preferred_element_type=jnp.float32, precision=_HI)
    out_ref[...] = h * dis_ref[...]


def _tc_mid(accp, ht, dis, b, W):
    return pl.pallas_call(
        _tc_mid_body,
        out_shape=jax.ShapeDtypeStruct((N, D), jnp.float32),
    )(accp, ht, dis, b.reshape(1, D), W)


def _tc_final_body(accp_ref, ht_ref, dis_ref, b_ref, batch_ref, out_ref):
    ht = ht_ref[...]
    padm = (lax.broadcasted_iota(jnp.int32, (N, 1), 0) < NPADE)
    agg = accp_ref[0] + accp_ref[1] + jnp.where(padm, 0.0, ht)
    z = jnp.maximum(dis_ref[...] * agg + b_ref[...], 0.0)
    seg = lax.broadcasted_iota(jnp.int32, (N, G), 1)
    onehot = (seg == batch_ref[...]).astype(jnp.float32)
    pool = lax.dot_general(onehot, z, (((0,), (0,)), ((), ())),
                           preferred_element_type=jnp.float32,
                           precision=_HI)
    out_ref[...] = pool


def _tc_final(accp, ht, dis, b, batch):
    return pl.pallas_call(
        _tc_final_body,
        out_shape=jax.ShapeDtypeStruct((G, D), jnp.float32),
    )(accp, ht, dis, b.reshape(1, D), batch.reshape(N, 1))


def kernel(x, edge_index, batch, W1, b1, W2, b2):
    pad = jnp.arange(NPADE, dtype=jnp.int32)
    src = jnp.concatenate([edge_index[0], pad])
    dst = jnp.concatenate([edge_index[1], pad])

    degp = _sc_deg(dst)
    dis, ht1 = _tc1(x, W1, degp)
    acc1 = _sc_agg(ht1, src, dst)
    ht2 = _tc_mid(acc1, ht1, dis, b1, W2)
    acc2 = _sc_agg(ht2, src, dst)
    return _tc_final(acc2, ht2, dis, b2, batch)


# speedup vs baseline: 1.0117x; 1.0117x over previous
import functools

import jax
import jax.numpy as jnp
from jax import lax
from jax.experimental import pallas as pl
from jax.experimental.pallas import tpu as pltpu
from jax.experimental.pallas import tpu_sc as plsc

N = 10000
E = 320000
D = 128
G = 64

NC = 2
NS = 16
NW = NC * NS

CHUNK = 128
NCHUNKS = E // CHUNK
CPT = 80
E_PER_TILE = CPT * CHUNK
E_PAD = NW * E_PER_TILE
NPADE = E_PAD - E
PHASE = 40
FIRE = 8

ROW_STEP = 624
ROW_SPAN = 640


def _zero_vmem_2d(ref, nrows):
    zero = jnp.zeros((16,), jnp.float32)
    ncols = ref.shape[1]

    def body(i, c):
        for u in range(ncols // 16):
            ref[i, pl.ds(u * 16, 16)] = zero
        return c

    lax.fori_loop(0, nrows, body, 0)


def _fill_ones_vmem_2d(ref, nrows):
    one = jnp.ones((16,), jnp.float32)
    ncols = ref.shape[1]

    def body(i, c):
        for u in range(ncols // 16):
            ref[i, pl.ds(u * 16, 16)] = one
        return c

    lax.fori_loop(0, nrows, body, 0)


def _zero_spmem_slice(acc_sh, row0, nrows, zbuf, zrows):
    nfull = nrows // zrows
    rem = nrows - nfull * zrows
    for k in range(nfull):
        pltpu.sync_copy(zbuf, acc_sh.at[pl.ds(row0 + k * zrows, zrows)])
    if rem:
        pltpu.sync_copy(zbuf.at[pl.ds(0, rem)],
                        acc_sh.at[pl.ds(row0 + nfull * zrows, rem)])


def _stage_idx(idx_hbm, estart, idx2, sem):
    def round_(ko, c):
        for b in range(FIRE):
            j = ko * FIRE + b
            pltpu.async_copy(idx_hbm.at[pl.ds(estart + j * CHUNK, CHUNK)],
                             idx2.at[j], sem)
        for b in range(FIRE):
            j = ko * FIRE + b
            pltpu.make_async_copy(
                idx_hbm.at[pl.ds(estart + j * CHUNK, CHUNK)],
                idx2.at[j], sem).wait()
        return c

    lax.fori_loop(0, idx2.shape[0] // FIRE, round_, 0)


def _sc_deg_body(dst_hbm, out_hbm, ones_v, zbuf, didx2, acc_sh, semi, sems):
    cid = lax.axis_index("c")
    sid = lax.axis_index("s")
    wid = sid * NC + cid

    _zero_vmem_2d(zbuf, CHUNK)
    _zero_spmem_slice(acc_sh, sid * ROW_STEP, ROW_SPAN, zbuf, CHUNK)
    _fill_ones_vmem_2d(ones_v, CHUNK)

    estart = wid * E_PER_TILE
    _stage_idx(dst_hbm, estart, didx2, semi)
    plsc.subcore_barrier()

    def round_(ko, c):
        for b in range(FIRE):
            j = ko * FIRE + b
            pltpu.async_copy(ones_v, acc_sh.at[didx2.at[j]], sems, add=True)
        for b in range(FIRE):
            j = ko * FIRE + b
            pltpu.make_async_copy(ones_v, acc_sh.at[didx2.at[j]],
                                  sems).wait()
        return c

    lax.fori_loop(0, CPT // FIRE, round_, 0)

    plsc.subcore_barrier()
    row0 = sid * ROW_STEP
    pltpu.sync_copy(acc_sh.at[pl.ds(row0, ROW_SPAN)],
                    out_hbm.at[cid, pl.ds(row0, ROW_SPAN)])


def _sc_deg(dst):
    mesh = plsc.VectorSubcoreMesh(core_axis_name="c", subcore_axis_name="s")
    k = functools.partial(
        pl.kernel,
        out_type=jax.ShapeDtypeStruct((NC, N, 16), jnp.float32),
        mesh=mesh,
        scratch_types=[
            pltpu.VMEM((CHUNK, 16), jnp.float32),
            pltpu.VMEM((CHUNK, 16), jnp.float32),
            pltpu.VMEM((CPT, CHUNK), jnp.int32),
            pltpu.VMEM_SHARED((N, 16), jnp.float32),
            pltpu.SemaphoreType.DMA,
            pltpu.SemaphoreType.DMA,
        ],
    )(_sc_deg_body)
    return k(dst)


def _sc_agg_body(ht_hbm, src_hbm, dst_hbm, out_hbm,
                 sidxA, didxA, sidxB, didxB, rows0, rows1,
                 acc_sh, sem0, sem1):
    cid = lax.axis_index("c")
    sid = lax.axis_index("s")
    wid = cid * NS + sid

    _zero_vmem_2d(rows0, CHUNK)
    _zero_spmem_slice(acc_sh, sid * ROW_STEP, ROW_SPAN, rows0, CHUNK)
    plsc.subcore_barrier()

    estart = wid * E_PER_TILE

    def load_idx(j, sidx, didx):
        pltpu.sync_copy(src_hbm.at[pl.ds(estart + j * CHUNK, CHUNK)], sidx)
        pltpu.sync_copy(dst_hbm.at[pl.ds(estart + j * CHUNK, CHUNK)], didx)

    def start_g(sidx, buf, sem):
        pltpu.async_copy(ht_hbm.at[sidx], buf, sem)

    def wait_g(sidx, buf, sem):
        pltpu.make_async_copy(ht_hbm.at[sidx], buf, sem).wait()

    def scat(didx, buf):
        pltpu.sync_copy(buf, acc_sh.at[didx], add=True)

    load_idx(0, sidxA, didxA)
    start_g(sidxA, rows0, sem0)

    def body(k, c):
        j1 = 2 * k + 1
        load_idx(j1, sidxB, didxB)
        start_g(sidxB, rows1, sem1)
        wait_g(sidxA, rows0, sem0)
        scat(didxA, rows0)
        load_idx(j1 + 1, sidxA, didxA)
        start_g(sidxA, rows0, sem0)
        wait_g(sidxB, rows1, sem1)
        scat(didxB, rows1)
        return c

    lax.fori_loop(0, CPT // 2 - 1, body, 0)

    j1 = CPT - 1
    load_idx(j1, sidxB, didxB)
    start_g(sidxB, rows1, sem1)
    wait_g(sidxA, rows0, sem0)
    scat(didxA, rows0)
    wait_g(sidxB, rows1, sem1)
    scat(didxB, rows1)

    plsc.subcore_barrier()
    row0 = sid * ROW_STEP
    pltpu.sync_copy(acc_sh.at[pl.ds(row0, ROW_SPAN)],
                    out_hbm.at[cid, pl.ds(row0, ROW_SPAN)])


def _sc_agg(ht, src, dst):
    mesh = plsc.VectorSubcoreMesh(core_axis_name="c", subcore_axis_name="s")
    k = functools.partial(
        pl.kernel,
        out_type=jax.ShapeDtypeStruct((NC, N, D), jnp.float32),
        mesh=mesh,
        scratch_types=[
            pltpu.VMEM((CHUNK,), jnp.int32),
            pltpu.VMEM((CHUNK,), jnp.int32),
            pltpu.VMEM((CHUNK,), jnp.int32),
            pltpu.VMEM((CHUNK,), jnp.int32),
            pltpu.VMEM((CHUNK, D), jnp.float32),
            pltpu.VMEM((CHUNK, D), jnp.float32),
            pltpu.VMEM_SHARED((N, D), jnp.float32),
            pltpu.SemaphoreType.DMA,
            pltpu.SemaphoreType.DMA,
        ],
    )(_sc_agg_body)
    return k(ht, src, dst)


_HI = jax.lax.Precision.HIGHEST


def _tc1_body(x_ref, w_ref, degp_ref, dis_ref, ht_ref):
    degp = degp_ref[...]
    padc = (lax.broadcasted_iota(jnp.int32, (N, 1), 0) < NPADE)
    deg = (degp[0, :, 0:1] + degp[1, :, 0:1] + 1.0
           - padc.astype(jnp.float32))
    dis = lax.rsqrt(deg)
    h = jnp.dot(x_ref[...], w_ref[...],
                preferred_element_type=jnp.float32, precision=_HI)
    dis_ref[...] = dis
    ht_ref[...] = h * dis


def _tc1(x, W1, degp):
    return pl.pallas_call(
        _tc1_body,
        out_shape=[
            jax.ShapeDtypeStruct((N, 1), jnp.float32),
            jax.ShapeDtypeStruct((N, D), jnp.float32),
        ],
    )(x, W1, degp)


def _tc_mid_body(accp_ref, ht_ref, dis_ref, b_ref, w_ref, out_ref):
    ht = ht_ref[...]
    padm = (lax.broadcasted_iota(jnp.int32, (N, 1), 0) < NPADE)
    selfloop = jnp.where(padm, 0.0, ht)
    agg = accp_ref[0] + accp_ref[1] + selfloop
    z = jnp.maximum(dis_ref[...] * agg + b_ref[...], 0.0)
    h = jnp.dot(z, w_ref[...],
                preferred_element_type=jnp.float32, precision=_HI)
    out_ref[...] = h * dis_ref[...]


def _tc_mid(accp, ht, dis, b, W):
    return pl.pallas_call(
        _tc_mid_body,
        out_shape=jax.ShapeDtypeStruct((N, D), jnp.float32),
    )(accp, ht, dis, b.reshape(1, D), W)


def _tc_final_body(accp_ref, ht_ref, dis_ref, b_ref, batch_ref, out_ref):
    ht = ht_ref[...]
    padm = (lax.broadcasted_iota(jnp.int32, (N, 1), 0) < NPADE)
    agg = accp_ref[0] + accp_ref[1] + jnp.where(padm, 0.0, ht)
    z = jnp.maximum(dis_ref[...] * agg + b_ref[...], 0.0)
    seg = lax.broadcasted_iota(jnp.int32, (N, G), 1)
    onehot = (seg == batch_ref[...]).astype(jnp.float32)
    pool = lax.dot_general(onehot, z, (((0,), (0,)), ((), ())),
                           preferred_element_type=jnp.float32,
                           precision=_HI)
    out_ref[...] = pool


def _tc_final(accp, ht, dis, b, batch):
    return pl.pallas_call(
        _tc_final_body,
        out_shape=jax.ShapeDtypeStruct((G, D), jnp.float32),
    )(accp, ht, dis, b.reshape(1, D), batch.reshape(N, 1))


def kernel(x, edge_index, batch, W1, b1, W2, b2):
    pad = jnp.arange(NPADE, dtype=jnp.int32)
    src = jnp.concatenate([edge_index[0], pad])
    dst = jnp.concatenate([edge_index[1], pad])

    degp = _sc_deg(dst)
    dis, ht1 = _tc1(x, W1, degp)
    acc1 = _sc_agg(ht1, src, dst)
    ht2 = _tc_mid(acc1, ht1, dis, b1, W2)
    acc2 = _sc_agg(ht2, src, dst)
    return _tc_final(acc2, ht2, dis, b2, batch)
